# initial kernel scaffold (unmeasured)
import jax
import jax.numpy as jnp
from jax import lax
from jax.experimental import pallas as pl
from jax.experimental.pallas import tpu as pltpu

H = 16
DH = 128
DR = 32
SCALE = (DH + DR) ** -0.5


def kernel(x, Wdkv, Wuk, Wuv, Wq, Wqr, Wkr, Wo):
    B, S, D = x.shape
    DC_SH = Wdkv.shape[1]

    def body(x_ref, wdkv_ref, wuk_ref, wuv_ref, wq_ref, wqr_ref, wkr_ref,
             wo_ref, out_ref,
             c_mine, c_other, wuk_other, wuv_other,
             k_ref, v_ref, q_ref, qr_ref, kr_ref, o_ref,
             send_sems, recv_sems):
        my_x = lax.axis_index("x")
        my_y = lax.axis_index("y")
        nbr = (my_x, 1 - my_y)

        barrier = pltpu.get_barrier_semaphore()
        pl.semaphore_signal(barrier, inc=1, device_id=nbr,
                            device_id_type=pl.DeviceIdType.MESH)
        pl.semaphore_wait(barrier, 1)

        xm = x_ref[0]
        c_mine[...] = jnp.dot(xm, wdkv_ref[...],
                              preferred_element_type=jnp.float32)

        rdmas = []
        for i, (src, dst) in enumerate(
            [(c_mine, c_other), (wuk_ref, wuk_other), (wuv_ref, wuv_other)]
        ):
            r = pltpu.make_async_remote_copy(
                src_ref=src, dst_ref=dst,
                send_sem=send_sems.at[i], recv_sem=recv_sems.at[i],
                device_id=nbr, device_id_type=pl.DeviceIdType.MESH,
            )
            r.start()
            rdmas.append(r)

        q_ref[...] = jnp.dot(xm, wq_ref[...],
                             preferred_element_type=jnp.float32)
        qr_ref[...] = jnp.dot(xm, wqr_ref[...],
                              preferred_element_type=jnp.float32)
        kr_ref[...] = jnp.dot(xm, wkr_ref[...],
                              preferred_element_type=jnp.float32)

        for r in rdmas:
            r.wait()

        k_ref[...] = (
            jnp.dot(c_mine[...], wuk_ref[...],
                    preferred_element_type=jnp.float32)
            + jnp.dot(c_other[...], wuk_other[...],
                      preferred_element_type=jnp.float32)
        )
        v_ref[...] = (
            jnp.dot(c_mine[...], wuv_ref[...],
                    preferred_element_type=jnp.float32)
            + jnp.dot(c_other[...], wuv_other[...],
                      preferred_element_type=jnp.float32)
        )

        kr = kr_ref[...]
        for h in range(H):
            qh = q_ref[:, h * DH:(h + 1) * DH]
            kh = k_ref[:, h * DH:(h + 1) * DH]
            qrh = qr_ref[:, h * DR:(h + 1) * DR]
            s = (
                lax.dot_general(qh, kh, (((1,), (1,)), ((), ())),
                                preferred_element_type=jnp.float32)
                + lax.dot_general(qrh, kr, (((1,), (1,)), ((), ())),
                                  preferred_element_type=jnp.float32)
            ) * SCALE
            m = jnp.max(s, axis=1, keepdims=True)
            p = jnp.exp(s - m)
            p = p / jnp.sum(p, axis=1, keepdims=True)
            o_ref[:, h * DH:(h + 1) * DH] = jnp.dot(
                p, v_ref[:, h * DH:(h + 1) * DH],
                preferred_element_type=jnp.float32)
        out_ref[0] = jnp.dot(o_ref[...], wo_ref[...],
                             preferred_element_type=jnp.float32)

    return pl.pallas_call(
        body,
        out_shape=jax.ShapeDtypeStruct((B, S, D), jnp.float32),
        in_specs=[pl.BlockSpec(memory_space=pltpu.VMEM)] * 8,
        out_specs=pl.BlockSpec(memory_space=pltpu.VMEM),
        scratch_shapes=[
            pltpu.VMEM((S, DC_SH), jnp.float32),
            pltpu.VMEM((S, DC_SH), jnp.float32),
            pltpu.VMEM((DC_SH, D), jnp.float32),
            pltpu.VMEM((DC_SH, D), jnp.float32),
            pltpu.VMEM((S, D), jnp.float32),
            pltpu.VMEM((S, D), jnp.float32),
            pltpu.VMEM((S, D), jnp.float32),
            pltpu.VMEM((S, H * DR), jnp.float32),
            pltpu.VMEM((S, DR), jnp.float32),
            pltpu.VMEM((S, D), jnp.float32),
            pltpu.SemaphoreType.DMA((3,)),
            pltpu.SemaphoreType.DMA((3,)),
        ],
        compiler_params=pltpu.CompilerParams(collective_id=0),
    )(x, Wdkv, Wuk, Wuv, Wq, Wqr, Wkr, Wo)


# baseline (device time: 157160 ns/iter reference)
import jax
import jax.numpy as jnp
from jax import lax
from jax.experimental import pallas as pl
from jax.experimental.pallas import tpu as pltpu

H = 16
DH = 128
DR = 32
SCALE = (DH + DR) ** -0.5
F32 = jnp.float32


def _dot(a, b):
    return jnp.dot(a, b, preferred_element_type=F32)


def _kv_comm(xm, Wdkv, Wuk, Wuv):
    S, D = xm.shape
    DC_SH = Wdkv.shape[1]

    def body(x_ref, wdkv_ref, wuk_ref, wuv_ref, k_ref, v_ref,
             c_mine, c_other, wuk_other, wuv_other, send_sems, recv_sems):
        my_x = lax.axis_index("x")
        my_y = lax.axis_index("y")
        nbr = (my_x, 1 - my_y)

        barrier = pltpu.get_barrier_semaphore()
        pl.semaphore_signal(barrier, inc=1, device_id=nbr,
                            device_id_type=pl.DeviceIdType.MESH)
        pl.semaphore_wait(barrier, 1)

        c_mine[...] = _dot(x_ref[...], wdkv_ref[...])

        rdmas = []
        for i, (src, dst) in enumerate(
            [(c_mine, c_other), (wuk_ref, wuk_other), (wuv_ref, wuv_other)]
        ):
            r = pltpu.make_async_remote_copy(
                src_ref=src, dst_ref=dst,
                send_sem=send_sems.at[i], recv_sem=recv_sems.at[i],
                device_id=nbr, device_id_type=pl.DeviceIdType.MESH,
            )
            r.start()
            rdmas.append(r)
        for r in rdmas:
            r.wait()

        k_ref[...] = (_dot(c_mine[...], wuk_ref[...])
                      + _dot(c_other[...], wuk_other[...]))
        v_ref[...] = (_dot(c_mine[...], wuv_ref[...])
                      + _dot(c_other[...], wuv_other[...]))

    return pl.pallas_call(
        body,
        out_shape=(jax.ShapeDtypeStruct((S, D), F32),
                   jax.ShapeDtypeStruct((S, D), F32)),
        in_specs=[pl.BlockSpec(memory_space=pltpu.VMEM)] * 4,
        out_specs=(pl.BlockSpec(memory_space=pltpu.VMEM),
                   pl.BlockSpec(memory_space=pltpu.VMEM)),
        scratch_shapes=[
            pltpu.VMEM((S, DC_SH), F32),
            pltpu.VMEM((S, DC_SH), F32),
            pltpu.VMEM((DC_SH, D), F32),
            pltpu.VMEM((DC_SH, D), F32),
            pltpu.SemaphoreType.DMA((3,)),
            pltpu.SemaphoreType.DMA((3,)),
        ],
        compiler_params=pltpu.CompilerParams(collective_id=0),
    )(xm, Wdkv, Wuk, Wuv)


def _proj(xm, Wq, Wqr, Wkr):
    S, D = xm.shape

    def body(x_ref, wq_ref, wqr_ref, wkr_ref, q_ref, qr_ref, kr_ref):
        q_ref[...] = _dot(x_ref[...], wq_ref[...])
        qr_ref[...] = _dot(x_ref[...], wqr_ref[...])
        kr_ref[...] = _dot(x_ref[...], wkr_ref[...])

    return pl.pallas_call(
        body,
        out_shape=(jax.ShapeDtypeStruct((S, H * DH), F32),
                   jax.ShapeDtypeStruct((S, H * DR), F32),
                   jax.ShapeDtypeStruct((S, DR), F32)),
        in_specs=[pl.BlockSpec(memory_space=pltpu.VMEM)] * 4,
        out_specs=(pl.BlockSpec(memory_space=pltpu.VMEM),) * 3,
    )(xm, Wq, Wqr, Wkr)


def _attn(q, qr, kr, k, v):
    S = q.shape[0]

    def body(q_ref, qr_ref, kr_ref, k_ref, v_ref, o_ref):
        s = (
            lax.dot_general(q_ref[...], k_ref[...], (((1,), (1,)), ((), ())),
                            preferred_element_type=F32)
            + lax.dot_general(qr_ref[0], kr_ref[...],
                              (((1,), (1,)), ((), ())),
                              preferred_element_type=F32)
        ) * SCALE
        m = jnp.max(s, axis=1, keepdims=True)
        p = jnp.exp(s - m)
        p = p / jnp.sum(p, axis=1, keepdims=True)
        o_ref[...] = _dot(p, v_ref[...])

    return pl.pallas_call(
        body,
        grid=(H,),
        out_shape=jax.ShapeDtypeStruct((S, H * DH), F32),
        in_specs=[
            pl.BlockSpec((S, DH), lambda h: (0, h)),
            pl.BlockSpec((1, S, DR), lambda h: (h, 0, 0)),
            pl.BlockSpec((S, DR), lambda h: (0, 0)),
            pl.BlockSpec((S, DH), lambda h: (0, h)),
            pl.BlockSpec((S, DH), lambda h: (0, h)),
        ],
        out_specs=pl.BlockSpec((S, DH), lambda h: (0, h)),
    )(q, qr, kr, k, v)


def _outproj(o, Wo):
    S, D = o.shape[0], Wo.shape[1]

    def body(o_ref, wo_ref, out_ref):
        out_ref[...] = _dot(o_ref[...], wo_ref[...])

    return pl.pallas_call(
        body,
        out_shape=jax.ShapeDtypeStruct((S, D), F32),
        in_specs=[pl.BlockSpec(memory_space=pltpu.VMEM)] * 2,
        out_specs=pl.BlockSpec(memory_space=pltpu.VMEM),
    )(o, Wo)


def kernel(x, Wdkv, Wuk, Wuv, Wq, Wqr, Wkr, Wo):
    B, S, D = x.shape
    xm = x.reshape(S, D)
    k, v = _kv_comm(xm, Wdkv, Wuk, Wuv)
    q, qr, kr = _proj(xm, Wq, Wqr, Wkr)
    qr_h = qr.reshape(S, H, DR).transpose(1, 0, 2)
    o = _attn(q, qr_h, kr, k, v)
    out = _outproj(o, Wo)
    return out.reshape(B, S, D)
